# baseline (speedup 1.0000x reference)
import functools

import jax
import jax.numpy as jnp
from jax import lax
from jax.experimental import pallas as pl
from jax.experimental.pallas import tpu as pltpu
from jax.experimental.pallas import tpu_sc as plsc

N = 100000
D = 128
S = 512
OUT = 7
ROWW = 144
NC = 2
NS = 16
NW = NC * NS

N_SC = 81920
CHUNK = N_SC // NW
BLK = 256
NBLK = CHUNK // BLK
GRP = BLK // 16
BWIN = 2576
BWIN_DMA = 2568

TCB = 1024
TC_OFF = N_SC // TCB
TC_GRID = (N - N_SC + TCB - 1) // TCB


def _sc_partial(h, batch):
    mesh = plsc.VectorSubcoreMesh(core_axis_name="c", subcore_axis_name="s")

    @functools.partial(
        pl.kernel,
        mesh=mesh,
        compiler_params=pltpu.CompilerParams(
            use_tc_tiling_on_sc=False, needs_layout_passes=False
        ),
        out_type=jax.ShapeDtypeStruct((NC, S, ROWW), jnp.float32),
        scratch_types=[
            pltpu.VMEM((BLK, D), jnp.float32),
            pltpu.VMEM((BLK, D), jnp.float32),
            pltpu.SemaphoreType.DMA,
            pltpu.SemaphoreType.DMA,
            pltpu.VMEM((BWIN,), jnp.int32),
            pltpu.VMEM((1, ROWW), jnp.float32),
            pltpu.VMEM((1,), jnp.int32),
            pltpu.VMEM((S // NS, ROWW), jnp.float32),
            pltpu.VMEM_SHARED((S, ROWW), jnp.float32),
        ],
    )
    def k(h_hbm, b_hbm, out_hbm, hbuf0, hbuf1, sem0, sem1, bwin, fvec, fidx,
          obuf, acc_sh):
        c = lax.axis_index("c")
        s = lax.axis_index("s")
        wid = c * NS + s
        start = wid * CHUNK
        rows_per_tile = S // NS

        z16 = jnp.zeros((16,), jnp.float32)
        for r in range(rows_per_tile):
            orow = obuf.at[r]
            for kk in range(ROWW // 16):
                orow[pl.ds(kk * 16, 16)] = z16
        pltpu.sync_copy(obuf, acc_sh.at[pl.ds(s * rows_per_tile, rows_per_tile)])
        plsc.subcore_barrier()

        a = pl.multiple_of((start // 8) * 8, 8)
        dlt = start - a
        pltpu.sync_copy(b_hbm.at[pl.ds(a, BWIN_DMA)], bwin.at[pl.ds(0, BWIN_DMA)])

        iota16 = lax.iota(jnp.int32, 16)

        def flush(cur, cnt, accs):
            frow = fvec.at[0]
            for kk in range(8):
                frow[pl.ds(kk * 16, 16)] = accs[kk]
            cvec = jnp.where(iota16 == 0, cnt.astype(jnp.float32), 0.0)
            frow[pl.ds(128, 16)] = cvec
            plsc.store_scatter(
                fidx,
                [jnp.zeros((16,), jnp.int32)],
                jnp.full((16,), cur, jnp.int32),
                mask=iota16 == 0,
            )
            pltpu.sync_copy(fvec, acc_sh.at[fidx], add=True)

        def h_desc(blk, buf, sem):
            return pltpu.make_async_copy(
                h_hbm.at[pl.ds(start + blk * BLK, BLK)], buf, sem
            )

        def row_step(base, buf, r, carry):
            cur, cnt = carry[0], carry[1]
            accs = carry[2:]
            seg = bwin[pl.ds(base + r, 16)][0]
            fl = seg != cur

            @pl.when(fl)
            def _():
                flush(cur, cnt, accs)

            row = buf.at[r]
            new_accs = tuple(
                jnp.where(fl, 0.0, accs[kk]) + row[pl.ds(kk * 16, 16)]
                for kk in range(8)
            )
            cnt = jnp.where(fl, 0, cnt) + 1
            return (seg, cnt) + new_accs

        def run_block(base, buf, carry):
            def group_body(g, carry):
                cur, cnt = carry[0], carry[1]
                accs = carry[2:]
                bvec = bwin[pl.ds(base + g * 16, 16)]
                b0 = bvec[0]
                uniform = b0 == bvec[15]

                def fast():
                    fl = b0 != cur

                    @pl.when(fl)
                    def _():
                        flush(cur, cnt, accs)

                    rows = [buf.at[g * 16 + rr] for rr in range(16)]
                    new_accs = []
                    for kk in range(8):
                        t = rows[0][pl.ds(kk * 16, 16)]
                        for rr in range(1, 16):
                            t = t + rows[rr][pl.ds(kk * 16, 16)]
                        new_accs.append(jnp.where(fl, 0.0, accs[kk]) + t)
                    ncnt = jnp.where(fl, 0, cnt) + 16
                    return (b0, ncnt) + tuple(new_accs)

                def slow():
                    return lax.fori_loop(
                        g * 16,
                        g * 16 + 16,
                        lambda r, cc: row_step(base, buf, r, cc),
                        carry,
                    )

                return lax.cond(uniform, fast, slow)

            return lax.fori_loop(0, GRP, group_body, carry)

        zero8 = tuple(jnp.zeros((16,), jnp.float32) for _ in range(8))
        carry = (bwin[pl.ds(dlt, 16)][0], jnp.int32(0)) + zero8

        h_desc(0, hbuf0, sem0).start()

        def pair_body(g, carry):
            b0 = 2 * g
            h_desc(b0 + 1, hbuf1, sem1).start()
            h_desc(b0, hbuf0, sem0).wait()
            carry = run_block(dlt + b0 * BLK, hbuf0, carry)
            h_desc(b0 + 2, hbuf0, sem0).start()
            h_desc(b0 + 1, hbuf1, sem1).wait()
            return run_block(dlt + (b0 + 1) * BLK, hbuf1, carry)

        carry = lax.fori_loop(0, (NBLK - 2) // 2, pair_body, carry)
        h_desc(NBLK - 1, hbuf1, sem1).start()
        h_desc(NBLK - 2, hbuf0, sem0).wait()
        carry = run_block(dlt + (NBLK - 2) * BLK, hbuf0, carry)
        h_desc(NBLK - 1, hbuf1, sem1).wait()
        carry = run_block(dlt + (NBLK - 1) * BLK, hbuf1, carry)
        flush(carry[0], carry[1], carry[2:])

        plsc.subcore_barrier()

        pltpu.sync_copy(acc_sh.at[pl.ds(s * rows_per_tile, rows_per_tile)], obuf)
        pltpu.sync_copy(obuf, out_hbm.at[c, pl.ds(s * rows_per_tile, rows_per_tile)])

    return k(h, batch)


def _tc_partial(h, batch):
    def k(b_ref, h_ref, sum_ref, cnt_ref, acc, cac):
        i = pl.program_id(0)

        @pl.when(i == 0)
        def _():
            acc[...] = jnp.zeros((S, D), jnp.float32)
            cac[...] = jnp.zeros((S, 1), jnp.float32)

        base = (TC_OFF + i) * TCB
        valid = (base + lax.broadcasted_iota(jnp.int32, (1, TCB), 1)) < N
        ids = b_ref[...].reshape(1, TCB)
        seg_iota = lax.broadcasted_iota(jnp.int32, (S, TCB), 0)
        onehot = jnp.where(
            jnp.logical_and(ids == seg_iota, valid), 1.0, 0.0
        ).astype(jnp.bfloat16)

        hb = jnp.where(
            jnp.transpose(valid, (1, 0)), h_ref[...], 0.0
        ).astype(jnp.bfloat16)
        acc[...] += lax.dot_general(
            onehot, hb, (((1,), (0,)), ((), ())),
            preferred_element_type=jnp.float32,
        )
        cac[...] += jnp.sum(
            onehot.astype(jnp.float32), axis=1, keepdims=True
        )

        @pl.when(i == TC_GRID - 1)
        def _():
            sum_ref[...] = acc[...]
            cnt_ref[...] = cac[...]

    return pl.pallas_call(
        k,
        grid=(TC_GRID,),
        in_specs=[
            pl.BlockSpec((TCB,), lambda i: (TC_OFF + i,)),
            pl.BlockSpec((TCB, D), lambda i: (TC_OFF + i, 0)),
        ],
        out_specs=[
            pl.BlockSpec((S, D), lambda i: (0, 0)),
            pl.BlockSpec((S, 1), lambda i: (0, 0)),
        ],
        out_shape=[
            jax.ShapeDtypeStruct((S, D), jnp.float32),
            jax.ShapeDtypeStruct((S, 1), jnp.float32),
        ],
        scratch_shapes=[
            pltpu.VMEM((S, D), jnp.float32),
            pltpu.VMEM((S, 1), jnp.float32),
        ],
    )(batch, h)


def _tc_finish(partial, tc_sum, tc_cnt, W, b):
    def k(p_ref, ts_ref, tc_ref, w_ref, b_ref, o_ref):
        p = p_ref[0] + p_ref[1]
        sums = p[:, :D] + ts_ref[...]
        cnt = p[:, D] + tc_ref[...][:, 0]
        mean = sums / jnp.maximum(cnt, 1.0)[:, None]
        o_ref[...] = (
            lax.dot_general(
                mean,
                w_ref[...],
                (((1,), (1,)), ((), ())),
                preferred_element_type=jnp.float32,
            )
            + b_ref[...]
        )

    return pl.pallas_call(
        k,
        out_shape=jax.ShapeDtypeStruct((S, OUT), jnp.float32),
    )(partial, tc_sum, tc_cnt, W, b.reshape(1, OUT))


def kernel(h, batch, W, b):
    partial = _sc_partial(h, batch)
    tc_sum, tc_cnt = _tc_partial(h, batch)
    return _tc_finish(partial, tc_sum, tc_cnt, W, b)


# speedup vs baseline: 1.0553x; 1.0553x over previous
import functools

import jax
import jax.numpy as jnp
from jax import lax
from jax.experimental import pallas as pl
from jax.experimental.pallas import tpu as pltpu
from jax.experimental.pallas import tpu_sc as plsc

N = 100000
D = 128
S = 512
OUT = 7
ROWW = 144
NC = 2
NS = 16
NW = NC * NS

N_SC = 76800
CHUNK = N_SC // NW
BLK = 240
NBLK = CHUNK // BLK
GRP = BLK // 16
BWIN = 2432
BWIN_DMA = 2408

TCB = 1024
TC_OFF = N_SC // TCB
TC_GRID = (N - N_SC + TCB - 1) // TCB


def _sc_partial(h, batch):
    mesh = plsc.VectorSubcoreMesh(core_axis_name="c", subcore_axis_name="s")

    @functools.partial(
        pl.kernel,
        mesh=mesh,
        compiler_params=pltpu.CompilerParams(
            use_tc_tiling_on_sc=False, needs_layout_passes=False
        ),
        out_type=jax.ShapeDtypeStruct((NC, S, ROWW), jnp.float32),
        scratch_types=[
            pltpu.VMEM((BLK, D), jnp.float32),
            pltpu.VMEM((BLK, D), jnp.float32),
            pltpu.SemaphoreType.DMA,
            pltpu.SemaphoreType.DMA,
            pltpu.VMEM((BWIN,), jnp.int32),
            pltpu.VMEM((1, ROWW), jnp.float32),
            pltpu.VMEM((1,), jnp.int32),
            pltpu.VMEM((S // NS, ROWW), jnp.float32),
            pltpu.VMEM_SHARED((S, ROWW), jnp.float32),
        ],
    )
    def k(h_hbm, b_hbm, out_hbm, hbuf0, hbuf1, sem0, sem1, bwin, fvec, fidx,
          obuf, acc_sh):
        c = lax.axis_index("c")
        s = lax.axis_index("s")
        wid = c * NS + s
        start = wid * CHUNK
        rows_per_tile = S // NS

        z16 = jnp.zeros((16,), jnp.float32)
        for r in range(rows_per_tile):
            orow = obuf.at[r]
            for kk in range(ROWW // 16):
                orow[pl.ds(kk * 16, 16)] = z16
        pltpu.sync_copy(obuf, acc_sh.at[pl.ds(s * rows_per_tile, rows_per_tile)])
        plsc.subcore_barrier()

        a = pl.multiple_of((start // 8) * 8, 8)
        dlt = start - a
        pltpu.sync_copy(b_hbm.at[pl.ds(a, BWIN_DMA)], bwin.at[pl.ds(0, BWIN_DMA)])

        iota16 = lax.iota(jnp.int32, 16)

        def flush(cur, cnt, accs):
            frow = fvec.at[0]
            for kk in range(8):
                frow[pl.ds(kk * 16, 16)] = accs[kk]
            cvec = jnp.where(iota16 == 0, cnt.astype(jnp.float32), 0.0)
            frow[pl.ds(128, 16)] = cvec
            plsc.store_scatter(
                fidx,
                [jnp.zeros((16,), jnp.int32)],
                jnp.full((16,), cur, jnp.int32),
                mask=iota16 == 0,
            )
            pltpu.sync_copy(fvec, acc_sh.at[fidx], add=True)

        def h_desc(blk, buf, sem):
            return pltpu.make_async_copy(
                h_hbm.at[pl.ds(start + blk * BLK, BLK)], buf, sem
            )

        def row_step(base, buf, r, carry):
            cur, cnt = carry[0], carry[1]
            accs = carry[2:]
            seg = bwin[pl.ds(base + r, 16)][0]
            fl = seg != cur

            @pl.when(fl)
            def _():
                flush(cur, cnt, accs)

            row = buf.at[r]
            new_accs = tuple(
                jnp.where(fl, 0.0, accs[kk]) + row[pl.ds(kk * 16, 16)]
                for kk in range(8)
            )
            cnt = jnp.where(fl, 0, cnt) + 1
            return (seg, cnt) + new_accs

        def run_block(base, buf, carry):
            def group_body(g, carry):
                cur, cnt = carry[0], carry[1]
                accs = carry[2:]
                bvec = bwin[pl.ds(base + g * 16, 16)]
                b0 = bvec[0]
                uniform = b0 == bvec[15]

                def fast():
                    fl = b0 != cur

                    @pl.when(fl)
                    def _():
                        flush(cur, cnt, accs)

                    rows = [buf.at[g * 16 + rr] for rr in range(16)]
                    new_accs = []
                    for kk in range(8):
                        t = rows[0][pl.ds(kk * 16, 16)]
                        for rr in range(1, 16):
                            t = t + rows[rr][pl.ds(kk * 16, 16)]
                        new_accs.append(jnp.where(fl, 0.0, accs[kk]) + t)
                    ncnt = jnp.where(fl, 0, cnt) + 16
                    return (b0, ncnt) + tuple(new_accs)

                def slow():
                    return lax.fori_loop(
                        g * 16,
                        g * 16 + 16,
                        lambda r, cc: row_step(base, buf, r, cc),
                        carry,
                    )

                return lax.cond(uniform, fast, slow)

            return lax.fori_loop(0, GRP, group_body, carry)

        zero8 = tuple(jnp.zeros((16,), jnp.float32) for _ in range(8))
        carry = (bwin[pl.ds(dlt, 16)][0], jnp.int32(0)) + zero8

        h_desc(0, hbuf0, sem0).start()

        def pair_body(g, carry):
            b0 = 2 * g
            h_desc(b0 + 1, hbuf1, sem1).start()
            h_desc(b0, hbuf0, sem0).wait()
            carry = run_block(dlt + b0 * BLK, hbuf0, carry)
            h_desc(b0 + 2, hbuf0, sem0).start()
            h_desc(b0 + 1, hbuf1, sem1).wait()
            return run_block(dlt + (b0 + 1) * BLK, hbuf1, carry)

        carry = lax.fori_loop(0, (NBLK - 2) // 2, pair_body, carry)
        h_desc(NBLK - 1, hbuf1, sem1).start()
        h_desc(NBLK - 2, hbuf0, sem0).wait()
        carry = run_block(dlt + (NBLK - 2) * BLK, hbuf0, carry)
        h_desc(NBLK - 1, hbuf1, sem1).wait()
        carry = run_block(dlt + (NBLK - 1) * BLK, hbuf1, carry)
        flush(carry[0], carry[1], carry[2:])

        plsc.subcore_barrier()

        pltpu.sync_copy(acc_sh.at[pl.ds(s * rows_per_tile, rows_per_tile)], obuf)
        pltpu.sync_copy(obuf, out_hbm.at[c, pl.ds(s * rows_per_tile, rows_per_tile)])

    return k(h, batch)


def _tc_partial(h, batch):
    def k(b_ref, h_ref, sum_ref, cnt_ref, acc, cac):
        i = pl.program_id(0)

        @pl.when(i == 0)
        def _():
            acc[...] = jnp.zeros((S, D), jnp.float32)
            cac[...] = jnp.zeros((S, 1), jnp.float32)

        base = (TC_OFF + i) * TCB
        valid = (base + lax.broadcasted_iota(jnp.int32, (1, TCB), 1)) < N
        ids = b_ref[...].reshape(1, TCB)
        seg_iota = lax.broadcasted_iota(jnp.int32, (S, TCB), 0)
        onehot = jnp.where(
            jnp.logical_and(ids == seg_iota, valid), 1.0, 0.0
        ).astype(jnp.bfloat16)

        hb = jnp.where(
            jnp.transpose(valid, (1, 0)), h_ref[...], 0.0
        ).astype(jnp.bfloat16)
        acc[...] += lax.dot_general(
            onehot, hb, (((1,), (0,)), ((), ())),
            preferred_element_type=jnp.float32,
        )
        cac[...] += jnp.sum(
            onehot.astype(jnp.float32), axis=1, keepdims=True
        )

        @pl.when(i == TC_GRID - 1)
        def _():
            sum_ref[...] = acc[...]
            cnt_ref[...] = cac[...]

    return pl.pallas_call(
        k,
        grid=(TC_GRID,),
        in_specs=[
            pl.BlockSpec((TCB,), lambda i: (TC_OFF + i,)),
            pl.BlockSpec((TCB, D), lambda i: (TC_OFF + i, 0)),
        ],
        out_specs=[
            pl.BlockSpec((S, D), lambda i: (0, 0)),
            pl.BlockSpec((S, 1), lambda i: (0, 0)),
        ],
        out_shape=[
            jax.ShapeDtypeStruct((S, D), jnp.float32),
            jax.ShapeDtypeStruct((S, 1), jnp.float32),
        ],
        scratch_shapes=[
            pltpu.VMEM((S, D), jnp.float32),
            pltpu.VMEM((S, 1), jnp.float32),
        ],
    )(batch, h)


def _tc_finish(partial, tc_sum, tc_cnt, W, b):
    def k(p_ref, ts_ref, tc_ref, w_ref, b_ref, o_ref):
        p = p_ref[0] + p_ref[1]
        sums = p[:, :D] + ts_ref[...]
        cnt = p[:, D] + tc_ref[...][:, 0]
        mean = sums / jnp.maximum(cnt, 1.0)[:, None]
        o_ref[...] = (
            lax.dot_general(
                mean,
                w_ref[...],
                (((1,), (1,)), ((), ())),
                preferred_element_type=jnp.float32,
            )
            + b_ref[...]
        )

    return pl.pallas_call(
        k,
        out_shape=jax.ShapeDtypeStruct((S, OUT), jnp.float32),
    )(partial, tc_sum, tc_cnt, W, b.reshape(1, OUT))


def kernel(h, batch, W, b):
    partial = _sc_partial(h, batch)
    tc_sum, tc_cnt = _tc_partial(h, batch)
    return _tc_finish(partial, tc_sum, tc_cnt, W, b)
